# baseline (speedup 1.0000x reference)
import jax
import jax.numpy as jnp
from jax import lax
from jax.experimental import pallas as pl
from jax.experimental.pallas import tpu as pltpu
from jax.experimental.pallas import tpu_sc as plsc

N_NODES = 10000
N_EDGES = 320000
IN_CH = 128
HID = 64
OUT_CH = 10
CH = 16
K = 3
LAMBDA_AMP = 0.5
GAMMA = 1.0 / (2.0 * (1.0 - LAMBDA_AMP))
G2 = GAMMA * 2.0 * (1.0 - LAMBDA_AMP)
LAM_EFF = GAMMA * LAMBDA_AMP

NS = 16
CHUNK = 128
NCHUNK = 160
NB = 8
NG = NCHUNK // NB
EPT = NCHUNK * CHUNK
EPAD = EPT * NS
NPAD = 10112
SPT = NPAD // NS


def _rsqrt16(x):
    i = plsc.bitcast(x, jnp.int32)
    i = jnp.int32(0x5F3759DF) - lax.shift_right_arithmetic(i, 1)
    r = plsc.bitcast(i, jnp.float32)
    for _ in range(3):
        r = r * (1.5 - 0.5 * x * r * r)
    return r


def _sc_fused_body(h_hbm, row_hbm, col_hbm, ones_hbm, zeros_hbm,
                   xk_hbm, u_hbm,
                   idx_r, idx_c, msg, hbuf, xbuf, dbuf, abuf, ubuf,
                   acc, sg, ss):
    s = lax.axis_index("s")
    stripe = pl.ds(s * SPT, SPT)

    pltpu.sync_copy(row_hbm.at[pl.ds(s * NCHUNK, NCHUNK)], idx_r)
    pltpu.sync_copy(col_hbm.at[pl.ds(s * NCHUNK, NCHUNK)], idx_c)
    pltpu.sync_copy(h_hbm.at[stripe], hbuf)
    pltpu.sync_copy(h_hbm.at[stripe], xbuf)
    pltpu.sync_copy(ones_hbm, msg.at[0])
    pltpu.sync_copy(zeros_hbm.at[stripe], acc.at[stripe])
    plsc.subcore_barrier()

    def deg_group(g, carry):
        sd = [
            pltpu.async_copy(msg.at[0], acc.at[idx_c.at[g * NB + b]], ss,
                             add=True)
            for b in range(NB)
        ]
        for d in sd:
            d.wait()
        return carry

    lax.fori_loop(0, NG, deg_group, 0)
    plsc.subcore_barrier()

    pltpu.sync_copy(acc.at[stripe], abuf)

    def dinv_row(r, carry):
        dinv = _rsqrt16(abuf[r, :] + 1.0)
        dbuf[r, :] = dinv
        ubuf[r, :] = dinv * hbuf[r, :]
        return carry

    lax.fori_loop(0, SPT, dinv_row, 0)
    pltpu.sync_copy(ubuf, u_hbm.at[stripe])
    plsc.subcore_barrier()

    def prop_iter(k, carry):
        pltpu.sync_copy(zeros_hbm.at[stripe], acc.at[stripe])
        plsc.subcore_barrier()

        def edge_group(g, carry2):
            gd = [
                pltpu.async_copy(u_hbm.at[idx_r.at[g * NB + b]], msg.at[b], sg)
                for b in range(NB)
            ]
            for d in gd:
                d.wait()
            sd = [
                pltpu.async_copy(msg.at[b], acc.at[idx_c.at[g * NB + b]], ss,
                                 add=True)
                for b in range(NB)
            ]
            for d in sd:
                d.wait()
            return carry2

        lax.fori_loop(0, NG, edge_group, 0)
        plsc.subcore_barrier()

        pltpu.sync_copy(acc.at[stripe], abuf)

        def prox_row(r, carry2):
            accr = abuf[r, :]
            xk = xbuf[r, :]
            h = hbuf[r, :]
            dinv = dbuf[r, :]
            y = (1.0 - G2) * xk + G2 * (dinv * accr + dinv * dinv * xk)
            d = y - h
            rn2 = jnp.sum(d * d)
            rn2v = jnp.maximum(jnp.zeros((CH,), jnp.float32) + rn2, 1e-30)
            scale = jnp.maximum(1.0 - LAM_EFF * _rsqrt16(rn2v), 0.0)
            xknew = h + scale * d
            xbuf[r, :] = xknew
            ubuf[r, :] = dinv * xknew
            return carry2

        lax.fori_loop(0, SPT, prox_row, 0)
        pltpu.sync_copy(ubuf, u_hbm.at[stripe])
        plsc.subcore_barrier()
        return carry

    lax.fori_loop(0, K, prop_iter, 0)
    pltpu.sync_copy(xbuf, xk_hbm.at[stripe])


_SC_MESH = plsc.VectorSubcoreMesh(core_axis_name="c", subcore_axis_name="s",
                                  num_cores=1)
_SC_PARAMS = pltpu.CompilerParams(use_tc_tiling_on_sc=False,
                                  needs_layout_passes=False)

_fused_sc = pl.kernel(
    _sc_fused_body,
    out_type=(jax.ShapeDtypeStruct((NPAD, CH), jnp.float32),
              jax.ShapeDtypeStruct((NPAD, CH), jnp.float32)),
    mesh=_SC_MESH,
    compiler_params=_SC_PARAMS,
    scratch_types=[
        pltpu.VMEM((NCHUNK, CHUNK), jnp.int32),
        pltpu.VMEM((NCHUNK, CHUNK), jnp.int32),
        pltpu.VMEM((NB, CHUNK, CH), jnp.float32),
        pltpu.VMEM((SPT, CH), jnp.float32),
        pltpu.VMEM((SPT, CH), jnp.float32),
        pltpu.VMEM((SPT, CH), jnp.float32),
        pltpu.VMEM((SPT, CH), jnp.float32),
        pltpu.VMEM((SPT, CH), jnp.float32),
        pltpu.VMEM_SHARED((NPAD, CH), jnp.float32),
        pltpu.SemaphoreType.DMA,
        pltpu.SemaphoreType.DMA,
    ],
)


def _mlp_body(x_ref, w1_ref, b1_ref, w2_ref, b2_ref, h_ref):
    h1 = jnp.dot(x_ref[...], w1_ref[...], preferred_element_type=jnp.float32)
    h1 = jnp.maximum(h1 + b1_ref[...], 0.0)
    h_ref[...] = jnp.dot(h1, w2_ref[...],
                         preferred_element_type=jnp.float32) + b2_ref[...]


_mlp = pl.pallas_call(
    _mlp_body,
    out_shape=jax.ShapeDtypeStruct((N_NODES, CH), jnp.float32),
)


def kernel(x, edge_index, W1, b1, W2, b2):
    ei = edge_index.astype(jnp.int32)
    row = jnp.pad(ei[0], (0, EPAD - N_EDGES)).reshape(EPAD // CHUNK, CHUNK)
    col = jnp.pad(ei[1], (0, EPAD - N_EDGES),
                  constant_values=N_NODES).reshape(EPAD // CHUNK, CHUNK)
    w2p = jnp.pad(W2, ((0, 0), (0, CH - OUT_CH)))
    b2p = jnp.pad(b2, (0, CH - OUT_CH)).reshape(1, CH)
    b1r = b1.reshape(1, HID)
    zeros = jnp.zeros((NPAD, CH), jnp.float32)
    ones = jnp.ones((CHUNK, CH), jnp.float32)

    h = jnp.pad(_mlp(x, W1, b1r, w2p, b2p), ((0, NPAD - N_NODES), (0, 0)))
    xk, _ = _fused_sc(h, row, col, ones, zeros)
    return xk[:N_NODES, :OUT_CH]


# speedup vs baseline: 1.2406x; 1.2406x over previous
import jax
import jax.numpy as jnp
from jax import lax
from jax.experimental import pallas as pl
from jax.experimental.pallas import tpu as pltpu
from jax.experimental.pallas import tpu_sc as plsc

N_NODES = 10000
N_EDGES = 320000
IN_CH = 128
HID = 64
OUT_CH = 10
CH = 16
K = 3
LAMBDA_AMP = 0.5
GAMMA = 1.0 / (2.0 * (1.0 - LAMBDA_AMP))
G2 = GAMMA * 2.0 * (1.0 - LAMBDA_AMP)
LAM_EFF = GAMMA * LAMBDA_AMP

NC = 2
NS = 16
NW = NC * NS
CHUNK = 128
NCHUNK = 80
NB = 4
NG = NCHUNK // NB
NBD = 8
NGD = NCHUNK // NBD
EPT = NCHUNK * CHUNK
EPAD = EPT * NW
NPAD = 10112
SPT = NPAD // NS


def _sc_deg_body(col_hbm, ones_hbm, zeros_hbm, out_hbm, idx_c, msg, acc, ss):
    c = lax.axis_index("c")
    s = lax.axis_index("s")
    w = s * NC + c
    pltpu.sync_copy(ones_hbm, msg)
    pltpu.sync_copy(col_hbm.at[pl.ds(w * NCHUNK, NCHUNK)], idx_c)
    pltpu.sync_copy(zeros_hbm.at[pl.ds(s * SPT, SPT)], acc.at[pl.ds(s * SPT, SPT)])
    plsc.subcore_barrier()

    for b in range(NBD):
        pltpu.async_copy(msg, acc.at[idx_c.at[b]], ss, add=True)

    def body(t, carry):
        for b in range(NBD):
            pltpu.async_copy(msg, acc.at[idx_c.at[(t + 1) * NBD + b]], ss,
                             add=True)
        for b in range(NBD):
            pltpu.make_async_copy(msg, acc.at[idx_c.at[t * NBD + b]], ss).wait()
        return carry

    lax.fori_loop(0, NGD - 1, body, 0)
    for b in range(NBD):
        pltpu.make_async_copy(msg, acc.at[idx_c.at[(NGD - 1) * NBD + b]],
                              ss).wait()
    plsc.subcore_barrier()
    pltpu.sync_copy(acc.at[pl.ds(s * SPT, SPT)],
                    out_hbm.at[c, pl.ds(s * SPT, SPT)])


def _sc_prop_body(u_hbm, row_hbm, col_hbm, zeros_hbm, out_hbm,
                  idx_r, idx_c, msg, acc, sg0, sg1, ss0, ss1):
    c = lax.axis_index("c")
    s = lax.axis_index("s")
    w = s * NC + c
    pltpu.sync_copy(row_hbm.at[pl.ds(w * NCHUNK, NCHUNK)],
                    idx_r.at[pl.ds(0, NCHUNK)])
    pltpu.sync_copy(row_hbm.at[pl.ds(w * NCHUNK, NB)],
                    idx_r.at[pl.ds(NCHUNK, NB)])
    pltpu.sync_copy(col_hbm.at[pl.ds(w * NCHUNK, NCHUNK)], idx_c)
    pltpu.sync_copy(zeros_hbm.at[pl.ds(s * SPT, SPT)], acc.at[pl.ds(s * SPT, SPT)])
    plsc.subcore_barrier()

    for b in range(NB):
        pltpu.async_copy(u_hbm.at[idx_r.at[b]], msg.at[0, b], sg0)

    def body(t, carry):
        g0 = 2 * t
        g1 = 2 * t + 1
        for b in range(NB):
            pltpu.make_async_copy(u_hbm.at[idx_r.at[g0 * NB + b]],
                                  msg.at[0, b], sg0).wait()
        for b in range(NB):
            pltpu.async_copy(u_hbm.at[idx_r.at[g1 * NB + b]], msg.at[1, b], sg1)
        for b in range(NB):
            pltpu.async_copy(msg.at[0, b], acc.at[idx_c.at[g0 * NB + b]], ss0,
                             add=True)
        for b in range(NB):
            pltpu.make_async_copy(u_hbm.at[idx_r.at[g1 * NB + b]],
                                  msg.at[1, b], sg1).wait()
        for b in range(NB):
            pltpu.make_async_copy(msg.at[0, b],
                                  acc.at[idx_c.at[g0 * NB + b]], ss0).wait()
        for b in range(NB):
            pltpu.async_copy(u_hbm.at[idx_r.at[(g0 + 2) * NB + b]],
                             msg.at[0, b], sg0)
        for b in range(NB):
            pltpu.async_copy(msg.at[1, b], acc.at[idx_c.at[g1 * NB + b]], ss1,
                             add=True)
        for b in range(NB):
            pltpu.make_async_copy(msg.at[1, b],
                                  acc.at[idx_c.at[g1 * NB + b]], ss1).wait()
        return carry

    lax.fori_loop(0, NG // 2, body, 0)
    for b in range(NB):
        pltpu.make_async_copy(u_hbm.at[idx_r.at[NCHUNK + b]],
                              msg.at[0, b], sg0).wait()
    plsc.subcore_barrier()
    pltpu.sync_copy(acc.at[pl.ds(s * SPT, SPT)],
                    out_hbm.at[c, pl.ds(s * SPT, SPT)])


_SC_MESH = plsc.VectorSubcoreMesh(core_axis_name="c", subcore_axis_name="s")
_SC_PARAMS = pltpu.CompilerParams(use_tc_tiling_on_sc=False)

_deg_sc = pl.kernel(
    _sc_deg_body,
    out_type=jax.ShapeDtypeStruct((NC, NPAD, CH), jnp.float32),
    mesh=_SC_MESH,
    compiler_params=_SC_PARAMS,
    scratch_types=[
        pltpu.VMEM((NCHUNK, CHUNK), jnp.int32),
        pltpu.VMEM((CHUNK, CH), jnp.float32),
        pltpu.VMEM_SHARED((NPAD, CH), jnp.float32),
        pltpu.SemaphoreType.DMA,
    ],
)

_prop_sc = pl.kernel(
    _sc_prop_body,
    out_type=jax.ShapeDtypeStruct((NC, NPAD, CH), jnp.float32),
    mesh=_SC_MESH,
    compiler_params=_SC_PARAMS,
    scratch_types=[
        pltpu.VMEM((NCHUNK + NB, CHUNK), jnp.int32),
        pltpu.VMEM((NCHUNK, CHUNK), jnp.int32),
        pltpu.VMEM((2, NB, CHUNK, CH), jnp.float32),
        pltpu.VMEM_SHARED((NPAD, CH), jnp.float32),
        pltpu.SemaphoreType.DMA,
        pltpu.SemaphoreType.DMA,
        pltpu.SemaphoreType.DMA,
        pltpu.SemaphoreType.DMA,
    ],
)


def _mlp_body(x_ref, w1_ref, b1_ref, w2_ref, b2_ref, h_ref):
    h1 = jnp.dot(x_ref[...], w1_ref[...], preferred_element_type=jnp.float32)
    h1 = jnp.maximum(h1 + b1_ref[...], 0.0)
    h_ref[...] = jnp.dot(h1, w2_ref[...],
                         preferred_element_type=jnp.float32) + b2_ref[...]


_mlp = pl.pallas_call(
    _mlp_body,
    out_shape=jax.ShapeDtypeStruct((N_NODES, CH), jnp.float32),
)


def _prep_body(dacc_ref, h_ref, dinv_ref, u_ref):
    dacc = dacc_ref[...]
    deg = 1.0 + dacc[0] + dacc[1]
    dinv = lax.rsqrt(deg)
    dinv_ref[...] = dinv
    u_ref[...] = dinv * h_ref[...]


_prep = pl.pallas_call(
    _prep_body,
    out_shape=(jax.ShapeDtypeStruct((NPAD, CH), jnp.float32),
               jax.ShapeDtypeStruct((NPAD, CH), jnp.float32)),
)


def _step_body(acc_ref, xk_ref, h_ref, dinv_ref, xknew_ref, unew_ref):
    a = acc_ref[...]
    acc = a[0] + a[1]
    dinv = dinv_ref[...]
    xk = xk_ref[...]
    h = h_ref[...]
    y = (1.0 - G2) * xk + G2 * (dinv * acc + dinv * dinv * xk)
    d = y - h
    rn = jnp.sqrt(jnp.sum(d * d, axis=1, keepdims=True))
    scale = jnp.maximum(rn - LAM_EFF, 0.0) / jnp.maximum(rn, 0.5 * LAM_EFF)
    xknew = h + scale * d
    xknew_ref[...] = xknew
    unew_ref[...] = dinv * xknew


_step = pl.pallas_call(
    _step_body,
    out_shape=(jax.ShapeDtypeStruct((NPAD, CH), jnp.float32),
               jax.ShapeDtypeStruct((NPAD, CH), jnp.float32)),
)


def kernel(x, edge_index, W1, b1, W2, b2):
    ei = edge_index.astype(jnp.int32)
    row = jnp.pad(ei[0], (0, EPAD - N_EDGES)).reshape(EPAD // CHUNK, CHUNK)
    col = jnp.pad(ei[1], (0, EPAD - N_EDGES),
                  constant_values=N_NODES).reshape(EPAD // CHUNK, CHUNK)
    w2p = jnp.pad(W2, ((0, 0), (0, CH - OUT_CH)))
    b2p = jnp.pad(b2, (0, CH - OUT_CH)).reshape(1, CH)
    b1r = b1.reshape(1, HID)
    zeros = jnp.zeros((NPAD, CH), jnp.float32)
    ones = jnp.ones((CHUNK, CH), jnp.float32)

    h = jnp.pad(_mlp(x, W1, b1r, w2p, b2p), ((0, NPAD - N_NODES), (0, 0)))
    dacc = _deg_sc(col, ones, zeros)
    dinv, u = _prep(dacc, h)
    xk = h
    for _ in range(K):
        acc = _prop_sc(u, row, col, zeros)
        xk, u = _step(acc, xk, h, dinv)
    return xk[:N_NODES, :OUT_CH]


# speedup vs baseline: 1.6979x; 1.3686x over previous
import jax
import jax.numpy as jnp
import numpy as np
from jax import lax
from jax.experimental import pallas as pl
from jax.experimental.pallas import tpu as pltpu
from jax.experimental.pallas import tpu_sc as plsc

N_NODES = 10000
N_EDGES = 320000
IN_CH = 128
HID = 64
OUT_CH = 10
CH = 16
K = 3
LAMBDA_AMP = 0.5
GAMMA = 1.0 / (2.0 * (1.0 - LAMBDA_AMP))
G2 = GAMMA * 2.0 * (1.0 - LAMBDA_AMP)
LAM_EFF = GAMMA * LAMBDA_AMP

NC = 2
NS = 16
NW = NC * NS
CHUNK = 128
NCHUNK = 80
NB = 4
NG = NCHUNK // NB
NBD = 8
NGD = NCHUNK // NBD
EPT = NCHUNK * CHUNK
EPAD = EPT * NW
ECH = N_EDGES // CHUNK
ECHP = EPAD // CHUNK
NPAD = 10112
SPT = NPAD // NS
R8 = NPAD // 8
RX = N_NODES // 8

_BLK = np.kron(np.eye(8, dtype=np.float32), np.ones((CH, CH), np.float32))


def _sc_deg_body(ei_hbm, ones_hbm, zeros_hbm, out_hbm, idx_c, msg, acc, ss):
    c = lax.axis_index("c")
    s = lax.axis_index("s")
    w = s * NC + c
    pltpu.sync_copy(ones_hbm, msg)
    pltpu.sync_copy(ei_hbm.at[1, pl.ds(w * NCHUNK, NCHUNK)], idx_c)
    pltpu.sync_copy(zeros_hbm.at[pl.ds(s * SPT, SPT)], acc.at[pl.ds(s * SPT, SPT)])
    plsc.subcore_barrier()

    for b in range(NBD):
        pltpu.async_copy(msg, acc.at[idx_c.at[b]], ss, add=True)

    def body(t, carry):
        for b in range(NBD):
            pltpu.async_copy(msg, acc.at[idx_c.at[(t + 1) * NBD + b]], ss,
                             add=True)
        for b in range(NBD):
            pltpu.make_async_copy(msg, acc.at[idx_c.at[t * NBD + b]], ss).wait()
        return carry

    lax.fori_loop(0, NGD - 1, body, 0)
    for b in range(NBD):
        pltpu.make_async_copy(msg, acc.at[idx_c.at[(NGD - 1) * NBD + b]],
                              ss).wait()
    plsc.subcore_barrier()
    pltpu.sync_copy(acc.at[pl.ds(s * SPT, SPT)],
                    out_hbm.at[c, pl.ds(s * SPT, SPT)])


def _sc_prop_body(u_hbm, ei_hbm, zeros_hbm, out_hbm,
                  idx_r, idx_c, msg, acc, sg0, sg1, ss0, ss1):
    c = lax.axis_index("c")
    s = lax.axis_index("s")
    w = s * NC + c
    pltpu.sync_copy(ei_hbm.at[0, pl.ds(w * NCHUNK, NCHUNK)],
                    idx_r.at[pl.ds(0, NCHUNK)])
    pltpu.sync_copy(ei_hbm.at[0, pl.ds(w * NCHUNK, NB)],
                    idx_r.at[pl.ds(NCHUNK, NB)])
    pltpu.sync_copy(ei_hbm.at[1, pl.ds(w * NCHUNK, NCHUNK)], idx_c)
    pltpu.sync_copy(zeros_hbm.at[pl.ds(s * SPT, SPT)], acc.at[pl.ds(s * SPT, SPT)])
    plsc.subcore_barrier()

    for b in range(NB):
        pltpu.async_copy(u_hbm.at[idx_r.at[b]], msg.at[0, b], sg0)

    def body(t, carry):
        g0 = 2 * t
        g1 = 2 * t + 1
        for b in range(NB):
            pltpu.make_async_copy(u_hbm.at[idx_r.at[g0 * NB + b]],
                                  msg.at[0, b], sg0).wait()
        for b in range(NB):
            pltpu.async_copy(u_hbm.at[idx_r.at[g1 * NB + b]], msg.at[1, b], sg1)
        for b in range(NB):
            pltpu.async_copy(msg.at[0, b], acc.at[idx_c.at[g0 * NB + b]], ss0,
                             add=True)
        for b in range(NB):
            pltpu.make_async_copy(u_hbm.at[idx_r.at[g1 * NB + b]],
                                  msg.at[1, b], sg1).wait()
        for b in range(NB):
            pltpu.make_async_copy(msg.at[0, b],
                                  acc.at[idx_c.at[g0 * NB + b]], ss0).wait()
        for b in range(NB):
            pltpu.async_copy(u_hbm.at[idx_r.at[(g0 + 2) * NB + b]],
                             msg.at[0, b], sg0)
        for b in range(NB):
            pltpu.async_copy(msg.at[1, b], acc.at[idx_c.at[g1 * NB + b]], ss1,
                             add=True)
        for b in range(NB):
            pltpu.make_async_copy(msg.at[1, b],
                                  acc.at[idx_c.at[g1 * NB + b]], ss1).wait()
        return carry

    lax.fori_loop(0, NG // 2, body, 0)
    for b in range(NB):
        pltpu.make_async_copy(u_hbm.at[idx_r.at[NCHUNK + b]],
                              msg.at[0, b], sg0).wait()
    plsc.subcore_barrier()
    pltpu.sync_copy(acc.at[pl.ds(s * SPT, SPT)],
                    out_hbm.at[c, pl.ds(s * SPT, SPT)])


_SC_MESH = plsc.VectorSubcoreMesh(core_axis_name="c", subcore_axis_name="s")
_SC_PARAMS = pltpu.CompilerParams(use_tc_tiling_on_sc=False)

_deg_sc = pl.kernel(
    _sc_deg_body,
    out_type=jax.ShapeDtypeStruct((NC, NPAD, CH), jnp.float32),
    mesh=_SC_MESH,
    compiler_params=_SC_PARAMS,
    scratch_types=[
        pltpu.VMEM((NCHUNK, CHUNK), jnp.int32),
        pltpu.VMEM((CHUNK, CH), jnp.float32),
        pltpu.VMEM_SHARED((NPAD, CH), jnp.float32),
        pltpu.SemaphoreType.DMA,
    ],
)

_prop_sc = pl.kernel(
    _sc_prop_body,
    out_type=jax.ShapeDtypeStruct((NC, NPAD, CH), jnp.float32),
    mesh=_SC_MESH,
    compiler_params=_SC_PARAMS,
    scratch_types=[
        pltpu.VMEM((NCHUNK + NB, CHUNK), jnp.int32),
        pltpu.VMEM((NCHUNK, CHUNK), jnp.int32),
        pltpu.VMEM((2, NB, CHUNK, CH), jnp.float32),
        pltpu.VMEM_SHARED((NPAD, CH), jnp.float32),
        pltpu.SemaphoreType.DMA,
        pltpu.SemaphoreType.DMA,
        pltpu.SemaphoreType.DMA,
        pltpu.SemaphoreType.DMA,
    ],
)


def _mlp_body(x8_ref, w1e_ref, b1e_ref, w2e_ref, b2e_ref, h_ref):
    h1 = jnp.dot(x8_ref[...], w1e_ref[...], preferred_element_type=jnp.float32)
    h1 = jnp.maximum(h1 + b1e_ref[...], 0.0)
    h2 = jnp.dot(h1, w2e_ref[...],
                 preferred_element_type=jnp.float32) + b2e_ref[...]
    h_ref[0:RX, :] = h2
    h_ref[RX:R8, :] = jnp.zeros((R8 - RX, 8 * CH), jnp.float32)


_mlp = pl.pallas_call(
    _mlp_body,
    out_shape=jax.ShapeDtypeStruct((R8, 8 * CH), jnp.float32),
)


def _prep_body(dacc_ref, h_ref, dinv_ref, u_ref):
    dacc = dacc_ref[...]
    dinv = lax.rsqrt(1.0 + dacc[0] + dacc[1])
    dinv_ref[...] = dinv
    u_ref[...] = dinv * h_ref[...]


_prep = pl.pallas_call(
    _prep_body,
    out_shape=(jax.ShapeDtypeStruct((R8, 8 * CH), jnp.float32),
               jax.ShapeDtypeStruct((R8, 8 * CH), jnp.float32)),
)


def _step_math(acc_ref, xk_ref, h_ref, dinv_ref, blk_ref):
    a = acc_ref[...]
    acc = a[0] + a[1]
    dinv = dinv_ref[...]
    xk = xk_ref[...]
    h = h_ref[...]
    y = (1.0 - G2) * xk + G2 * (dinv * acc + dinv * dinv * xk)
    d = y - h
    rn2 = jnp.dot(d * d, blk_ref[...], preferred_element_type=jnp.float32)
    scale = jnp.maximum(1.0 - LAM_EFF * lax.rsqrt(jnp.maximum(rn2, 1e-30)),
                        0.0)
    return h + scale * d, dinv


def _step_body(acc_ref, xk_ref, h_ref, dinv_ref, blk_ref, xknew_ref, unew_ref):
    xknew, dinv = _step_math(acc_ref, xk_ref, h_ref, dinv_ref, blk_ref)
    xknew_ref[...] = xknew
    unew_ref[...] = dinv * xknew


_step = pl.pallas_call(
    _step_body,
    out_shape=(jax.ShapeDtypeStruct((R8, 8 * CH), jnp.float32),
               jax.ShapeDtypeStruct((R8, 8 * CH), jnp.float32)),
)


def _stepf_body(acc_ref, xk_ref, h_ref, dinv_ref, blk_ref, xknew_ref):
    xknew, _ = _step_math(acc_ref, xk_ref, h_ref, dinv_ref, blk_ref)
    xknew_ref[...] = xknew


_stepf = pl.pallas_call(
    _stepf_body,
    out_shape=jax.ShapeDtypeStruct((R8, 8 * CH), jnp.float32),
)


def kernel(x, edge_index, W1, b1, W2, b2):
    f32 = jnp.float32
    ei3 = jnp.pad(edge_index.astype(jnp.int32).reshape(2, ECH, CHUNK),
                  ((0, 0), (0, ECHP - ECH), (0, 0)),
                  constant_values=N_NODES)

    w2p = jnp.pad(W2, ((0, 0), (0, CH - OUT_CH)))
    w1e = jnp.zeros((8 * IN_CH, 8 * HID), f32)
    w2e = jnp.zeros((8 * HID, 8 * CH), f32)
    for a in range(8):
        w1e = w1e.at[a * IN_CH:(a + 1) * IN_CH, a * HID:(a + 1) * HID].set(W1)
        w2e = w2e.at[a * HID:(a + 1) * HID, a * CH:(a + 1) * CH].set(w2p)
    b1e = jnp.tile(b1, 8).reshape(1, 8 * HID)
    b2e = jnp.tile(jnp.pad(b2, (0, CH - OUT_CH)), 8).reshape(1, 8 * CH)
    x8 = x.reshape(RX, 8 * IN_CH)
    blk = jnp.asarray(_BLK)
    zeros = jnp.zeros((NPAD, CH), f32)
    ones = jnp.ones((CHUNK, CH), f32)

    hp = _mlp(x8, w1e, b1e, w2e, b2e)
    dacc = _deg_sc(ei3, ones, zeros)
    dinvp, up = _prep(dacc.reshape(NC, R8, 8 * CH), hp)
    xkp = hp
    for k in range(K):
        acc = _prop_sc(up.reshape(NPAD, CH), ei3, zeros)
        accp = acc.reshape(NC, R8, 8 * CH)
        if k < K - 1:
            xkp, up = _step(accp, xkp, hp, dinvp, blk)
        else:
            xkp = _stepf(accp, xkp, hp, dinvp, blk)
    return xkp.reshape(NPAD, CH)[:N_NODES, :OUT_CH]


# speedup vs baseline: 2.6749x; 1.5754x over previous
import jax
import jax.numpy as jnp
import numpy as np
from jax import lax
from jax.experimental import pallas as pl
from jax.experimental.pallas import tpu as pltpu
from jax.experimental.pallas import tpu_sc as plsc

N_NODES = 10000
N_EDGES = 320000
IN_CH = 128
HID = 64
OUT_CH = 10
CH = 16
K = 3
LAMBDA_AMP = 0.5
GAMMA = 1.0 / (2.0 * (1.0 - LAMBDA_AMP))
G2 = GAMMA * 2.0 * (1.0 - LAMBDA_AMP)
LAM_EFF = GAMMA * LAMBDA_AMP

NC = 2
NS = 16
NW = NC * NS
CHUNK = 128
NCHUNK = 80
NB = 4
NG = NCHUNK // NB
NBD = 8
NGD = NCHUNK // NBD
EPT = NCHUNK * CHUNK
EPAD = EPT * NW
ECH = N_EDGES // CHUNK
ECHP = EPAD // CHUNK
NPAD = 10112
SPT = NPAD // NS
R8 = NPAD // 8
RX = N_NODES // 8

_BLK = np.kron(np.eye(8, dtype=np.float32), np.ones((CH, CH), np.float32))


def _sc_deg_body(ei_hbm, ones_hbm, zeros_hbm, out_hbm, idx_c, msg, acc, ss):
    c = lax.axis_index("c")
    s = lax.axis_index("s")
    w = s * NC + c
    pltpu.sync_copy(ones_hbm, msg)
    pltpu.sync_copy(ei_hbm.at[1, pl.ds(w * NCHUNK, NCHUNK)], idx_c)
    pltpu.sync_copy(zeros_hbm.at[pl.ds(s * SPT, SPT)], acc.at[pl.ds(s * SPT, SPT)])
    plsc.subcore_barrier()

    for b in range(NBD):
        pltpu.async_copy(msg, acc.at[idx_c.at[b]], ss, add=True)

    def body(t, carry):
        for b in range(NBD):
            pltpu.async_copy(msg, acc.at[idx_c.at[(t + 1) * NBD + b]], ss,
                             add=True)
        for b in range(NBD):
            pltpu.make_async_copy(msg, acc.at[idx_c.at[t * NBD + b]], ss).wait()
        return carry

    lax.fori_loop(0, NGD - 1, body, 0)
    for b in range(NBD):
        pltpu.make_async_copy(msg, acc.at[idx_c.at[(NGD - 1) * NBD + b]],
                              ss).wait()
    plsc.subcore_barrier()
    pltpu.sync_copy(acc.at[pl.ds(s * SPT, SPT)],
                    out_hbm.at[c, pl.ds(s * SPT, SPT)])


def _sc_prop_body(u_hbm, ei_hbm, zeros_hbm, out_hbm,
                  idx_r, idx_c, msg, u_sh, acc, sg0, sg1, ss0, ss1):
    c = lax.axis_index("c")
    s = lax.axis_index("s")
    w = s * NC + c
    pltpu.sync_copy(ei_hbm.at[0, pl.ds(w * NCHUNK, NCHUNK)],
                    idx_r.at[pl.ds(0, NCHUNK)])
    pltpu.sync_copy(ei_hbm.at[0, pl.ds(w * NCHUNK, NB)],
                    idx_r.at[pl.ds(NCHUNK, NB)])
    pltpu.sync_copy(ei_hbm.at[1, pl.ds(w * NCHUNK, NCHUNK)], idx_c)
    pltpu.sync_copy(u_hbm.at[pl.ds(s * SPT, SPT)], u_sh.at[pl.ds(s * SPT, SPT)])
    pltpu.sync_copy(zeros_hbm.at[pl.ds(s * SPT, SPT)], acc.at[pl.ds(s * SPT, SPT)])
    plsc.subcore_barrier()

    for b in range(NB):
        pltpu.async_copy(u_sh.at[idx_r.at[b]], msg.at[0, b], sg0)

    def body(t, carry):
        g0 = 2 * t
        g1 = 2 * t + 1
        for b in range(NB):
            pltpu.make_async_copy(u_sh.at[idx_r.at[g0 * NB + b]],
                                  msg.at[0, b], sg0).wait()
        for b in range(NB):
            pltpu.async_copy(u_sh.at[idx_r.at[g1 * NB + b]], msg.at[1, b], sg1)
        for b in range(NB):
            pltpu.async_copy(msg.at[0, b], acc.at[idx_c.at[g0 * NB + b]], ss0,
                             add=True)
        for b in range(NB):
            pltpu.make_async_copy(u_sh.at[idx_r.at[g1 * NB + b]],
                                  msg.at[1, b], sg1).wait()
        for b in range(NB):
            pltpu.make_async_copy(msg.at[0, b],
                                  acc.at[idx_c.at[g0 * NB + b]], ss0).wait()
        for b in range(NB):
            pltpu.async_copy(u_sh.at[idx_r.at[(g0 + 2) * NB + b]],
                             msg.at[0, b], sg0)
        for b in range(NB):
            pltpu.async_copy(msg.at[1, b], acc.at[idx_c.at[g1 * NB + b]], ss1,
                             add=True)
        for b in range(NB):
            pltpu.make_async_copy(msg.at[1, b],
                                  acc.at[idx_c.at[g1 * NB + b]], ss1).wait()
        return carry

    lax.fori_loop(0, NG // 2, body, 0)
    for b in range(NB):
        pltpu.make_async_copy(u_sh.at[idx_r.at[NCHUNK + b]],
                              msg.at[0, b], sg0).wait()
    plsc.subcore_barrier()
    pltpu.sync_copy(acc.at[pl.ds(s * SPT, SPT)],
                    out_hbm.at[c, pl.ds(s * SPT, SPT)])


_SC_MESH = plsc.VectorSubcoreMesh(core_axis_name="c", subcore_axis_name="s")
_SC_PARAMS = pltpu.CompilerParams(use_tc_tiling_on_sc=False)

_deg_sc = pl.kernel(
    _sc_deg_body,
    out_type=jax.ShapeDtypeStruct((NC, NPAD, CH), jnp.float32),
    mesh=_SC_MESH,
    compiler_params=_SC_PARAMS,
    scratch_types=[
        pltpu.VMEM((NCHUNK, CHUNK), jnp.int32),
        pltpu.VMEM((CHUNK, CH), jnp.float32),
        pltpu.VMEM_SHARED((NPAD, CH), jnp.float32),
        pltpu.SemaphoreType.DMA,
    ],
)

_prop_sc = pl.kernel(
    _sc_prop_body,
    out_type=jax.ShapeDtypeStruct((NC, NPAD, CH), jnp.float32),
    mesh=_SC_MESH,
    compiler_params=_SC_PARAMS,
    scratch_types=[
        pltpu.VMEM((NCHUNK + NB, CHUNK), jnp.int32),
        pltpu.VMEM((NCHUNK, CHUNK), jnp.int32),
        pltpu.VMEM((2, NB, CHUNK, CH), jnp.float32),
        pltpu.VMEM_SHARED((NPAD, CH), jnp.float32),
        pltpu.VMEM_SHARED((NPAD, CH), jnp.float32),
        pltpu.SemaphoreType.DMA,
        pltpu.SemaphoreType.DMA,
        pltpu.SemaphoreType.DMA,
        pltpu.SemaphoreType.DMA,
    ],
)


def _mlp_body(x8_ref, w1e_ref, b1e_ref, w2e_ref, b2e_ref, h_ref):
    h1 = jnp.dot(x8_ref[...], w1e_ref[...], preferred_element_type=jnp.float32)
    h1 = jnp.maximum(h1 + b1e_ref[...], 0.0)
    h2 = jnp.dot(h1, w2e_ref[...],
                 preferred_element_type=jnp.float32) + b2e_ref[...]
    h_ref[0:RX, :] = h2
    h_ref[RX:R8, :] = jnp.zeros((R8 - RX, 8 * CH), jnp.float32)


_mlp = pl.pallas_call(
    _mlp_body,
    out_shape=jax.ShapeDtypeStruct((R8, 8 * CH), jnp.float32),
)


def _prep_body(dacc_ref, h_ref, dinv_ref, u_ref):
    dacc = dacc_ref[...]
    dinv = lax.rsqrt(1.0 + dacc[0] + dacc[1])
    dinv_ref[...] = dinv
    u_ref[...] = dinv * h_ref[...]


_prep = pl.pallas_call(
    _prep_body,
    out_shape=(jax.ShapeDtypeStruct((R8, 8 * CH), jnp.float32),
               jax.ShapeDtypeStruct((R8, 8 * CH), jnp.float32)),
)


def _step_math(acc_ref, xk_ref, h_ref, dinv_ref, blk_ref):
    a = acc_ref[...]
    acc = a[0] + a[1]
    dinv = dinv_ref[...]
    xk = xk_ref[...]
    h = h_ref[...]
    y = (1.0 - G2) * xk + G2 * (dinv * acc + dinv * dinv * xk)
    d = y - h
    rn2 = jnp.dot(d * d, blk_ref[...], preferred_element_type=jnp.float32)
    scale = jnp.maximum(1.0 - LAM_EFF * lax.rsqrt(jnp.maximum(rn2, 1e-30)),
                        0.0)
    return h + scale * d, dinv


def _step_body(acc_ref, xk_ref, h_ref, dinv_ref, blk_ref, xknew_ref, unew_ref):
    xknew, dinv = _step_math(acc_ref, xk_ref, h_ref, dinv_ref, blk_ref)
    xknew_ref[...] = xknew
    unew_ref[...] = dinv * xknew


_step = pl.pallas_call(
    _step_body,
    out_shape=(jax.ShapeDtypeStruct((R8, 8 * CH), jnp.float32),
               jax.ShapeDtypeStruct((R8, 8 * CH), jnp.float32)),
)


def _stepf_body(acc_ref, xk_ref, h_ref, dinv_ref, blk_ref, xknew_ref):
    xknew, _ = _step_math(acc_ref, xk_ref, h_ref, dinv_ref, blk_ref)
    xknew_ref[...] = xknew


_stepf = pl.pallas_call(
    _stepf_body,
    out_shape=jax.ShapeDtypeStruct((R8, 8 * CH), jnp.float32),
)


def kernel(x, edge_index, W1, b1, W2, b2):
    f32 = jnp.float32
    ei3 = jnp.pad(edge_index.astype(jnp.int32).reshape(2, ECH, CHUNK),
                  ((0, 0), (0, ECHP - ECH), (0, 0)),
                  constant_values=N_NODES)

    w2p = jnp.pad(W2, ((0, 0), (0, CH - OUT_CH)))
    w1e = jnp.zeros((8 * IN_CH, 8 * HID), f32)
    w2e = jnp.zeros((8 * HID, 8 * CH), f32)
    for a in range(8):
        w1e = w1e.at[a * IN_CH:(a + 1) * IN_CH, a * HID:(a + 1) * HID].set(W1)
        w2e = w2e.at[a * HID:(a + 1) * HID, a * CH:(a + 1) * CH].set(w2p)
    b1e = jnp.tile(b1, 8).reshape(1, 8 * HID)
    b2e = jnp.tile(jnp.pad(b2, (0, CH - OUT_CH)), 8).reshape(1, 8 * CH)
    x8 = x.reshape(RX, 8 * IN_CH)
    blk = jnp.asarray(_BLK)
    zeros = jnp.zeros((NPAD, CH), f32)
    ones = jnp.ones((CHUNK, CH), f32)

    hp = _mlp(x8, w1e, b1e, w2e, b2e)
    dacc = _deg_sc(ei3, ones, zeros)
    dinvp, up = _prep(dacc.reshape(NC, R8, 8 * CH), hp)
    xkp = hp
    for k in range(K):
        acc = _prop_sc(up.reshape(NPAD, CH), ei3, zeros)
        accp = acc.reshape(NC, R8, 8 * CH)
        if k < K - 1:
            xkp, up = _step(accp, xkp, hp, dinvp, blk)
        else:
            xkp = _stepf(accp, xkp, hp, dinvp, blk)
    return xkp.reshape(NPAD, CH)[:N_NODES, :OUT_CH]


# speedup vs baseline: 2.7932x; 1.0442x over previous
import jax
import jax.numpy as jnp
import numpy as np
from jax import lax
from jax.experimental import pallas as pl
from jax.experimental.pallas import tpu as pltpu
from jax.experimental.pallas import tpu_sc as plsc

N_NODES = 10000
N_EDGES = 320000
IN_CH = 128
HID = 64
OUT_CH = 10
CH = 16
K = 3
LAMBDA_AMP = 0.5
GAMMA = 1.0 / (2.0 * (1.0 - LAMBDA_AMP))
G2 = GAMMA * 2.0 * (1.0 - LAMBDA_AMP)
LAM_EFF = GAMMA * LAMBDA_AMP

NC = 2
NS = 16
NW = NC * NS
CHUNK = 128
NCHUNK = 80
NB = 4
NG = NCHUNK // NB
NBD = 8
NGD = NCHUNK // NBD
EPT = NCHUNK * CHUNK
EPAD = EPT * NW
ECH = N_EDGES // CHUNK
ECHP = EPAD // CHUNK
NPAD = 10112
SPT = NPAD // NS
R8 = NPAD // 8
RX = N_NODES // 8

_BLK = np.kron(np.eye(8, dtype=np.float32), np.ones((CH, CH), np.float32))


def _sc_deg_body(ei_hbm, ones_hbm, zeros_hbm, out_hbm, idx_c, msg, acc, ss):
    c = lax.axis_index("c")
    s = lax.axis_index("s")
    w = s * NC + c
    pltpu.sync_copy(ones_hbm, msg)
    pltpu.sync_copy(ei_hbm.at[1, pl.ds(w * NCHUNK, NCHUNK)], idx_c)
    pltpu.sync_copy(zeros_hbm.at[pl.ds(s * SPT, SPT)], acc.at[pl.ds(s * SPT, SPT)])
    plsc.subcore_barrier()

    for b in range(NBD):
        pltpu.async_copy(msg, acc.at[idx_c.at[b]], ss, add=True)

    def body(t, carry):
        for b in range(NBD):
            pltpu.async_copy(msg, acc.at[idx_c.at[(t + 1) * NBD + b]], ss,
                             add=True)
        for b in range(NBD):
            pltpu.make_async_copy(msg, acc.at[idx_c.at[t * NBD + b]], ss).wait()
        return carry

    lax.fori_loop(0, NGD - 1, body, 0)
    for b in range(NBD):
        pltpu.make_async_copy(msg, acc.at[idx_c.at[(NGD - 1) * NBD + b]],
                              ss).wait()
    plsc.subcore_barrier()
    pltpu.sync_copy(acc.at[pl.ds(s * SPT, SPT)],
                    out_hbm.at[c, pl.ds(s * SPT, SPT)])


def _sc_prop_body(u_hbm, ei_hbm, zeros_hbm, out_hbm,
                  idx_r, idx_c, msg, u_sh, acc, sg0, sg1, ss0, ss1):
    c = lax.axis_index("c")
    s = lax.axis_index("s")
    w = s * NC + c
    pltpu.sync_copy(ei_hbm.at[0, pl.ds(w * NCHUNK, NCHUNK)],
                    idx_r.at[pl.ds(0, NCHUNK)])
    pltpu.sync_copy(ei_hbm.at[0, pl.ds(w * NCHUNK, NB)],
                    idx_r.at[pl.ds(NCHUNK, NB)])
    pltpu.sync_copy(ei_hbm.at[1, pl.ds(w * NCHUNK, NCHUNK)], idx_c)
    pltpu.sync_copy(u_hbm.at[pl.ds(s * SPT, SPT)], u_sh.at[pl.ds(s * SPT, SPT)])
    pltpu.sync_copy(zeros_hbm.at[pl.ds(s * SPT, SPT)], acc.at[pl.ds(s * SPT, SPT)])
    plsc.subcore_barrier()

    for b in range(NB):
        pltpu.async_copy(u_sh.at[idx_r.at[b]], msg.at[0, b], sg0)

    def body(t, carry):
        g0 = 2 * t
        g1 = 2 * t + 1
        for b in range(NB):
            pltpu.make_async_copy(u_sh.at[idx_r.at[g0 * NB + b]],
                                  msg.at[0, b], sg0).wait()
        for b in range(NB):
            pltpu.async_copy(u_sh.at[idx_r.at[g1 * NB + b]], msg.at[1, b], sg1)
        for b in range(NB):
            pltpu.async_copy(msg.at[0, b], acc.at[idx_c.at[g0 * NB + b]], ss0,
                             add=True)
        for b in range(NB):
            pltpu.make_async_copy(u_sh.at[idx_r.at[g1 * NB + b]],
                                  msg.at[1, b], sg1).wait()
        for b in range(NB):
            pltpu.make_async_copy(msg.at[0, b],
                                  acc.at[idx_c.at[g0 * NB + b]], ss0).wait()
        for b in range(NB):
            pltpu.async_copy(u_sh.at[idx_r.at[(g0 + 2) * NB + b]],
                             msg.at[0, b], sg0)
        for b in range(NB):
            pltpu.async_copy(msg.at[1, b], acc.at[idx_c.at[g1 * NB + b]], ss1,
                             add=True)
        for b in range(NB):
            pltpu.make_async_copy(msg.at[1, b],
                                  acc.at[idx_c.at[g1 * NB + b]], ss1).wait()
        return carry

    lax.fori_loop(0, NG // 2, body, 0)
    for b in range(NB):
        pltpu.make_async_copy(u_sh.at[idx_r.at[NCHUNK + b]],
                              msg.at[0, b], sg0).wait()
    plsc.subcore_barrier()
    pltpu.sync_copy(acc.at[pl.ds(s * SPT, SPT)],
                    out_hbm.at[c, pl.ds(s * SPT, SPT)])


_SC_MESH = plsc.VectorSubcoreMesh(core_axis_name="c", subcore_axis_name="s")
_SC_PARAMS = pltpu.CompilerParams(use_tc_tiling_on_sc=False)

_deg_sc = pl.kernel(
    _sc_deg_body,
    out_type=jax.ShapeDtypeStruct((NC, NPAD, CH), jnp.float32),
    mesh=_SC_MESH,
    compiler_params=_SC_PARAMS,
    scratch_types=[
        pltpu.VMEM((NCHUNK, CHUNK), jnp.int32),
        pltpu.VMEM((CHUNK, CH), jnp.float32),
        pltpu.VMEM_SHARED((NPAD, CH), jnp.float32),
        pltpu.SemaphoreType.DMA,
    ],
)

_prop_sc = pl.kernel(
    _sc_prop_body,
    out_type=jax.ShapeDtypeStruct((NC, NPAD, CH), jnp.float32),
    mesh=_SC_MESH,
    compiler_params=_SC_PARAMS,
    scratch_types=[
        pltpu.VMEM((NCHUNK + NB, CHUNK), jnp.int32),
        pltpu.VMEM((NCHUNK, CHUNK), jnp.int32),
        pltpu.VMEM((2, NB, CHUNK, CH), jnp.float32),
        pltpu.VMEM_SHARED((NPAD, CH), jnp.float32),
        pltpu.VMEM_SHARED((NPAD, CH), jnp.float32),
        pltpu.SemaphoreType.DMA,
        pltpu.SemaphoreType.DMA,
        pltpu.SemaphoreType.DMA,
        pltpu.SemaphoreType.DMA,
    ],
)


def _mlp_body(x_ref, w1_ref, b1_ref, w2_ref, b2_ref, h_ref):
    for a in range(8):
        lo = a * R8
        na = min(R8, N_NODES - lo)
        h1 = jnp.dot(x_ref[lo:lo + na, :], w1_ref[...],
                     preferred_element_type=jnp.float32)
        h1 = jnp.maximum(h1 + b1_ref[...], 0.0)
        h2 = jnp.dot(h1, w2_ref[...],
                     preferred_element_type=jnp.float32) + b2_ref[...]
        h_ref[0:na, a * CH:(a + 1) * CH] = h2
        if na < R8:
            h_ref[na:R8, a * CH:(a + 1) * CH] = jnp.zeros(
                (R8 - na, CH), jnp.float32)


_mlp = pl.pallas_call(
    _mlp_body,
    out_shape=jax.ShapeDtypeStruct((R8, 8 * CH), jnp.float32),
)


def _prep_body(dacc_ref, h_ref, dinv_ref, u_ref):
    dacc = dacc_ref[...]
    dinv = lax.rsqrt(1.0 + dacc[0] + dacc[1])
    dinv_ref[...] = dinv
    u_ref[...] = dinv * h_ref[...]


_prep = pl.pallas_call(
    _prep_body,
    out_shape=(jax.ShapeDtypeStruct((R8, 8 * CH), jnp.float32),
               jax.ShapeDtypeStruct((R8, 8 * CH), jnp.float32)),
)


def _step_math(acc_ref, xk_ref, h_ref, dinv_ref, blk_ref):
    a = acc_ref[...]
    acc = a[0] + a[1]
    dinv = dinv_ref[...]
    xk = xk_ref[...]
    h = h_ref[...]
    y = (1.0 - G2) * xk + G2 * (dinv * acc + dinv * dinv * xk)
    d = y - h
    rn2 = jnp.dot(d * d, blk_ref[...], preferred_element_type=jnp.float32)
    scale = jnp.maximum(1.0 - LAM_EFF * lax.rsqrt(jnp.maximum(rn2, 1e-30)),
                        0.0)
    return h + scale * d, dinv


def _step_body(acc_ref, xk_ref, h_ref, dinv_ref, blk_ref, xknew_ref, unew_ref):
    xknew, dinv = _step_math(acc_ref, xk_ref, h_ref, dinv_ref, blk_ref)
    xknew_ref[...] = xknew
    unew_ref[...] = dinv * xknew


_step = pl.pallas_call(
    _step_body,
    out_shape=(jax.ShapeDtypeStruct((R8, 8 * CH), jnp.float32),
               jax.ShapeDtypeStruct((R8, 8 * CH), jnp.float32)),
)


def _stepf_body(acc_ref, xk_ref, h_ref, dinv_ref, blk_ref, xknew_ref):
    xknew, _ = _step_math(acc_ref, xk_ref, h_ref, dinv_ref, blk_ref)
    xknew_ref[...] = xknew


_stepf = pl.pallas_call(
    _stepf_body,
    out_shape=jax.ShapeDtypeStruct((R8, 8 * CH), jnp.float32),
)


def kernel(x, edge_index, W1, b1, W2, b2):
    f32 = jnp.float32
    ei = edge_index.astype(jnp.int32)
    eip = (ei % R8) * 8 + ei // R8
    ei3 = jnp.pad(eip.reshape(2, ECH, CHUNK),
                  ((0, 0), (0, ECHP - ECH), (0, 0)),
                  constant_values=(N_NODES % R8) * 8 + N_NODES // R8)

    w2p = jnp.pad(W2, ((0, 0), (0, CH - OUT_CH)))
    b1r = b1.reshape(1, HID)
    b2p = jnp.pad(b2, (0, CH - OUT_CH)).reshape(1, CH)
    blk = jnp.asarray(_BLK)
    zeros = jnp.zeros((NPAD, CH), f32)
    ones = jnp.ones((CHUNK, CH), f32)

    hp = _mlp(x, W1, b1r, w2p, b2p)
    dacc = _deg_sc(ei3, ones, zeros)
    dinvp, up = _prep(dacc.reshape(NC, R8, 8 * CH), hp)
    xkp = hp
    for k in range(K):
        acc = _prop_sc(up.reshape(NPAD, CH), ei3, zeros)
        accp = acc.reshape(NC, R8, 8 * CH)
        if k < K - 1:
            xkp, up = _step(accp, xkp, hp, dinvp, blk)
        else:
            xkp = _stepf(accp, xkp, hp, dinvp, blk)
    out = xkp.reshape(R8, 8, CH).transpose(1, 0, 2).reshape(NPAD, CH)
    return out[:N_NODES, :OUT_CH]
